# Initial kernel scaffold; baseline (speedup 1.0000x reference)
import jax
import jax.numpy as jnp
from jax.experimental import pallas as pl


def kernel(updates, mask):
    raise NotImplementedError("write your pallas kernel here")


# speedup vs baseline: 4.4990x; 4.4990x over previous
import functools

import jax
import jax.numpy as jnp
from jax import lax
from jax.experimental import pallas as pl
from jax.experimental.pallas import tpu as pltpu
from jax.experimental.pallas import tpu_sc as plsc

B, H, W, C = 2, 256, 256, 96
N = B * H * W * C
OUT_FLAT = B * H * 2 * W * 2 * C

NC, NS = 2, 16
NW = NC * NS
LANES = 16

CHUNK = 3 * 2 ** 19
NCHUNK = OUT_FLAT // CHUNK
PASSES = NCHUNK // NC

PER_T = N // NS
STAGE = 8192
SITERS = PER_T // STAGE
WCH = CHUNK // NS
ZBUF = 8192
ZITERS = WCH // ZBUF


@functools.partial(
    pl.kernel,
    out_type=jax.ShapeDtypeStruct((OUT_FLAT,), jnp.float32),
    mesh=plsc.VectorSubcoreMesh(core_axis_name="c", subcore_axis_name="s"),
    scratch_types=[
        pltpu.VMEM((STAGE,), jnp.int32),
        pltpu.VMEM((STAGE,), jnp.float32),
        pltpu.VMEM((ZBUF,), jnp.float32),
        pltpu.VMEM_SHARED((CHUNK,), jnp.float32),
    ],
)
def _scatter_add_sc(upd_hbm, mask_hbm, out_hbm, idx_v, val_v, zbuf, acc):
    cid = lax.axis_index("c")
    sid = lax.axis_index("s")
    elt0 = sid * PER_T
    zvec = jnp.zeros((LANES,), jnp.float32)
    neg1 = jnp.full((LANES,), -1, jnp.int32)

    def zfill(i, carry):
        zbuf[pl.ds(i * LANES, LANES)] = zvec
        return carry
    lax.fori_loop(0, ZBUF // LANES, zfill, 0)

    def one_pass(p, carry):
        base = (p * NC + cid) * CHUNK

        for z in range(ZITERS):
            pltpu.sync_copy(zbuf, acc.at[pl.ds(sid * WCH + z * ZBUF, ZBUF)])
        plsc.subcore_barrier()

        def stage(it, carry2):
            off = elt0 + it * STAGE
            pltpu.sync_copy(mask_hbm.at[pl.ds(off, STAGE)], idx_v)
            pltpu.sync_copy(upd_hbm.at[pl.ds(off, STAGE)], val_v)

            def remap(r, carry3):
                for c in range(8):
                    sl = pl.ds((r * 8 + c) * LANES, LANES)
                    v = idx_v[sl]
                    rel = v - base
                    ok = (v >= base) & (rel < CHUNK)
                    idx_v[sl] = jnp.where(ok, rel, neg1)
                return carry3
            lax.fori_loop(0, STAGE // (8 * LANES), remap, 0)

            pltpu.sync_copy(
                val_v,
                acc.at[plsc.Indices(idx_v, ignored_value=-1)],
                add=True,
            )
            return carry2
        lax.fori_loop(0, SITERS, stage, 0)
        plsc.subcore_barrier()

        pltpu.sync_copy(acc.at[pl.ds(sid * WCH, WCH)],
                        out_hbm.at[pl.ds(base + sid * WCH, WCH)])
        return carry
    lax.fori_loop(0, PASSES, one_pass, 0)


def kernel(updates, mask):
    upd1 = updates.reshape(N)
    msk1 = mask.astype(jnp.int32).reshape(N)
    ret = _scatter_add_sc(upd1, msk1)
    return ret.reshape(B, H * 2, W * 2, C)


# speedup vs baseline: 4.5044x; 1.0012x over previous
import functools

import jax
import jax.numpy as jnp
from jax import lax
from jax.experimental import pallas as pl
from jax.experimental.pallas import tpu as pltpu
from jax.experimental.pallas import tpu_sc as plsc

B, H, W, C = 2, 256, 256, 96
N = B * H * W * C
OUT_FLAT = B * H * 2 * W * 2 * C

NC, NS = 2, 16
LANES = 16

CHUNK = 3 * 2 ** 19
NCHUNK = OUT_FLAT // CHUNK
PASSES = NCHUNK // NC

PER_T = N // NS
STAGE = 2048
SITERS = PER_T // STAGE
WCH = CHUNK // NS

BLK = 2048
CAPL = 640
HWML = CAPL - STAGE // LANES
CB = CAPL * LANES
NBLK = CB // BLK
TRASH = CB
CAP = CB + LANES


@functools.partial(
    pl.kernel,
    out_type=jax.ShapeDtypeStruct((OUT_FLAT,), jnp.float32),
    mesh=plsc.VectorSubcoreMesh(core_axis_name="c", subcore_axis_name="s"),
    compiler_params=pltpu.CompilerParams(needs_layout_passes=False),
    scratch_types=[
        pltpu.VMEM((STAGE,), jnp.int32),
        pltpu.VMEM((STAGE,), jnp.int32),
        pltpu.VMEM((STAGE,), jnp.float32),
        pltpu.VMEM((STAGE,), jnp.float32),
        pltpu.VMEM((CAP,), jnp.int32),
        pltpu.VMEM((CAP,), jnp.float32),
        pltpu.VMEM_SHARED((CHUNK,), jnp.float32),
        pltpu.SemaphoreType.DMA,
        pltpu.SemaphoreType.DMA,
    ],
)
def _scatter_add_sc(upd_hbm, mask_hbm, zero_hbm, out_hbm,
                    idx_a, idx_b, val_a, val_b, cb_idx, cb_val, acc,
                    sem_a, sem_b):
    cid = lax.axis_index("c")
    sid = lax.axis_index("s")
    elt0 = sid * PER_T
    iota = lax.iota(jnp.int32, LANES)
    trash = TRASH + iota
    neg1 = jnp.full((LANES,), -1, jnp.int32)
    ones = jnp.full((LANES,), 1, jnp.int32)
    zeros_i = jnp.zeros((LANES,), jnp.int32)

    bufs = ((idx_a, val_a, sem_a), (idx_b, val_b, sem_b))

    def stage_copies(it, bi, bv, sem):
        eoff = elt0 + it * STAGE
        return (
            pltpu.make_async_copy(mask_hbm.at[pl.ds(eoff, STAGE)], bi, sem),
            pltpu.make_async_copy(upd_hbm.at[pl.ds(eoff, STAGE)], bv, sem),
        )

    def cb_reset(j, carry):
        cb_idx[pl.ds(j * LANES, LANES)] = neg1
        return carry

    lax.fori_loop(0, CB // LANES, cb_reset, 0)

    def flush(mx):
        end = ((mx * LANES + BLK - 1) // BLK) * BLK
        for b in range(NBLK):
            @pl.when(b * BLK < end)
            def _():
                pltpu.sync_copy(
                    cb_val.at[pl.ds(b * BLK, BLK)],
                    acc.at[plsc.Indices(cb_idx.at[pl.ds(b * BLK, BLK)],
                                        ignored_value=-1)],
                    add=True,
                )
        lax.fori_loop(0, CB // LANES, cb_reset, 0)

    def one_pass(p, carry):
        base = (p * NC + cid) * CHUNK

        pltpu.sync_copy(zero_hbm, acc.at[pl.ds(sid * WCH, WCH)])
        plsc.subcore_barrier()

        c0, c1 = stage_copies(0, *bufs[0])
        c0.start(); c1.start()
        c2, c3 = stage_copies(1, *bufs[1])
        c2.start(); c3.start()

        def two_stages(g, cnt):
            for p2 in range(2):
                it = g * 2 + p2
                bi, bv, sem = bufs[p2]
                w0, w1 = stage_copies(it, bi, bv, sem)
                w0.wait(); w1.wait()

                def compact(r, cnt2):
                    for c in range(8):
                        sl = pl.ds((r * 8 + c) * LANES, LANES)
                        v = bi[sl]
                        w = bv[sl]
                        rel = v - base
                        ok = (v >= base) & (rel < CHUNK)
                        pos = jnp.where(ok, iota + (cnt2 << 4), trash)
                        plsc.store_scatter(cb_idx, [pos], rel)
                        plsc.store_scatter(cb_val, [pos], w)
                        cnt2 = cnt2 + jnp.where(ok, ones, zeros_i)
                    return cnt2
                cnt = lax.fori_loop(0, STAGE // (8 * LANES), compact, cnt)

                @pl.when(it + 2 < SITERS)
                def _():
                    s0, s1 = stage_copies(it + 2, bi, bv, sem)
                    s0.start(); s1.start()

                mx = jnp.max(cnt)
                do_flush = mx >= HWML
                @pl.when(do_flush)
                def _():
                    flush(mx)
                cnt = jnp.where(do_flush, zeros_i, cnt)
            return cnt

        cnt = lax.fori_loop(0, SITERS // 2, two_stages, zeros_i)
        flush(jnp.max(cnt))
        plsc.subcore_barrier()

        pltpu.sync_copy(acc.at[pl.ds(sid * WCH, WCH)],
                        out_hbm.at[pl.ds(base + sid * WCH, WCH)])
        return carry
    lax.fori_loop(0, PASSES, one_pass, 0)


def kernel(updates, mask):
    upd1 = updates.reshape(N)
    msk1 = mask.astype(jnp.int32).reshape(N)
    zero = jnp.zeros((WCH,), jnp.float32)
    ret = _scatter_add_sc(upd1, msk1, zero)
    return ret.reshape(B, H * 2, W * 2, C)


# speedup vs baseline: 4.8014x; 1.0659x over previous
import functools

import jax
import jax.numpy as jnp
from jax import lax
from jax.experimental import pallas as pl
from jax.experimental.pallas import tpu as pltpu
from jax.experimental.pallas import tpu_sc as plsc

B, H, W, C = 2, 256, 256, 96
N = B * H * W * C
OUT_FLAT = B * H * 2 * W * 2 * C

NC, NS = 2, 16
LANES = 16

CHUNK = 3 * 2 ** 19
NCHUNK = OUT_FLAT // CHUNK
PASSES = NCHUNK // NC

PER_T = N // NS
STAGE = 2048
SITERS = PER_T // STAGE
WCH = CHUNK // NS

BLK = 2048
CAPL = 640
HWML = CAPL - STAGE // LANES
CB = CAPL * LANES
NBLK = CB // BLK
TRASH = CB
CAP = CB + LANES


@functools.partial(
    pl.kernel,
    out_type=jax.ShapeDtypeStruct((OUT_FLAT,), jnp.float32),
    mesh=plsc.VectorSubcoreMesh(core_axis_name="c", subcore_axis_name="s"),
    compiler_params=pltpu.CompilerParams(needs_layout_passes=False),
    scratch_types=[
        pltpu.VMEM((STAGE,), jnp.int32),
        pltpu.VMEM((STAGE,), jnp.int32),
        pltpu.VMEM((STAGE,), jnp.float32),
        pltpu.VMEM((STAGE,), jnp.float32),
        pltpu.VMEM((CAP,), jnp.int32),
        pltpu.VMEM((CAP,), jnp.float32),
        pltpu.VMEM_SHARED((CHUNK,), jnp.float32),
        pltpu.SemaphoreType.DMA,
        pltpu.SemaphoreType.DMA,
    ],
)
def _scatter_add_sc(upd_hbm, mask_hbm, zero_hbm, out_hbm,
                    idx_a, idx_b, val_a, val_b, cb_idx, cb_val, acc,
                    sem_a, sem_b):
    cid = lax.axis_index("c")
    sid = lax.axis_index("s")
    elt0 = sid * PER_T
    iota = lax.iota(jnp.int32, LANES)
    neg1 = jnp.full((LANES,), -1, jnp.int32)
    sixteens = jnp.full((LANES,), LANES, jnp.int32)
    zeros_i = jnp.zeros((LANES,), jnp.int32)

    bufs = ((idx_a, val_a, sem_a), (idx_b, val_b, sem_b))

    def stage_copies(it, bi, bv, sem):
        eoff = elt0 + it * STAGE
        return (
            pltpu.make_async_copy(mask_hbm.at[pl.ds(eoff, STAGE)], bi, sem),
            pltpu.make_async_copy(upd_hbm.at[pl.ds(eoff, STAGE)], bv, sem),
        )

    def cb_reset(j, carry):
        cb_idx[pl.ds(j * LANES, LANES)] = neg1
        return carry

    lax.fori_loop(0, CB // LANES, cb_reset, 0)

    def flush(mx):
        end = ((mx + BLK - 1) // BLK) * BLK
        for b in range(NBLK):
            @pl.when(b * BLK < end)
            def _():
                pltpu.sync_copy(
                    cb_val.at[pl.ds(b * BLK, BLK)],
                    acc.at[plsc.Indices(cb_idx.at[pl.ds(b * BLK, BLK)],
                                        ignored_value=-1)],
                    add=True,
                )
        lax.fori_loop(0, CB // LANES, cb_reset, 0)

    def one_pass(p, carry):
        base = (p * NC + cid) * CHUNK

        pltpu.sync_copy(zero_hbm, acc.at[pl.ds(sid * WCH, WCH)])
        plsc.subcore_barrier()

        c0, c1 = stage_copies(0, *bufs[0])
        c0.start(); c1.start()
        c2, c3 = stage_copies(1, *bufs[1])
        c2.start(); c3.start()

        def two_stages(g, cnt):
            for p2 in range(2):
                it = g * 2 + p2
                bi, bv, sem = bufs[p2]
                w0, w1 = stage_copies(it, bi, bv, sem)
                w0.wait(); w1.wait()

                def compact(r, cnt2):
                    for c in range(16):
                        sl = pl.ds((r * 16 + c) * LANES, LANES)
                        v = bi[sl]
                        w = bv[sl]
                        rel = v - base
                        ok = lax.bitcast_convert_type(rel, jnp.uint32) < CHUNK
                        pos = iota + jnp.where(ok, cnt2, TRASH)
                        plsc.store_scatter(cb_idx, [pos], rel)
                        plsc.store_scatter(cb_val, [pos], w)
                        cnt2 = cnt2 + jnp.where(ok, sixteens, zeros_i)
                    return cnt2
                cnt = lax.fori_loop(0, STAGE // (16 * LANES), compact, cnt)

                @pl.when(it + 2 < SITERS)
                def _():
                    s0, s1 = stage_copies(it + 2, bi, bv, sem)
                    s0.start(); s1.start()

                mx = jnp.max(cnt)
                do_flush = mx >= HWML * LANES
                @pl.when(do_flush)
                def _():
                    flush(mx)
                cnt = jnp.where(do_flush, zeros_i, cnt)
            return cnt

        cnt = lax.fori_loop(0, SITERS // 2, two_stages, zeros_i)
        flush(jnp.max(cnt))
        plsc.subcore_barrier()

        pltpu.sync_copy(acc.at[pl.ds(sid * WCH, WCH)],
                        out_hbm.at[pl.ds(base + sid * WCH, WCH)])
        return carry
    lax.fori_loop(0, PASSES, one_pass, 0)


def kernel(updates, mask):
    upd1 = updates.reshape(N)
    msk1 = mask.astype(jnp.int32).reshape(N)
    zero = jnp.zeros((WCH,), jnp.float32)
    ret = _scatter_add_sc(upd1, msk1, zero)
    return ret.reshape(B, H * 2, W * 2, C)
